# baseline (speedup 1.0000x reference)
import functools

import jax
import jax.numpy as jnp
from jax import lax
from jax.experimental import pallas as pl
from jax.experimental.pallas import tpu as pltpu
from jax.experimental.pallas import tpu_sc as plsc

S = 2048
D = 1024
E = 8
FF = 1024
TOPK = 2
PAIRS = S * TOPK
TM = 256
NPR = PAIRS + E * TM
NB = NPR // TM
NBE = 64
NW = 32
L = 16


def _router_body(x_ref, wg_ref, d1_ref, d2_ref, w1_ref, w2_ref, be_ref):
    x = x_ref[...]
    logits = jnp.dot(x, wg_ref[...], preferred_element_type=jnp.float32)
    lane = lax.broadcasted_iota(jnp.int32, (S, E), 1)
    l1 = jnp.max(logits, axis=1, keepdims=True)
    i1 = jnp.min(jnp.where(logits == l1, lane, E), axis=1, keepdims=True)
    oh1 = lane == i1
    masked = jnp.where(oh1, -jnp.inf, logits)
    l2 = jnp.max(masked, axis=1, keepdims=True)
    i2 = jnp.min(jnp.where(masked == l2, lane, E), axis=1, keepdims=True)
    oh2 = lane == i2
    w1 = 1.0 / (1.0 + jnp.exp(l2 - l1))
    w1_ref[...] = w1
    w2_ref[...] = 1.0 - w1

    oh1f = oh1.astype(jnp.float32)
    oh2f = oh2.astype(jnp.float32)
    ohb = jnp.concatenate([oh1f, oh2f], axis=1)
    CH = 128
    r = lax.broadcasted_iota(jnp.int32, (CH, CH), 0)
    c = lax.broadcasted_iota(jnp.int32, (CH, CH), 1)
    tri = (c < r).astype(jnp.float32)
    csum = jnp.zeros((1, 2 * E), jnp.float32)
    parts = []
    for k in range(S // CH):
        seg = ohb[k * CH:(k + 1) * CH, :]
        parts.append(jnp.dot(tri, seg, preferred_element_type=jnp.float32)
                     + csum)
        csum = csum + jnp.sum(seg, axis=0, keepdims=True)
    excl = jnp.concatenate(parts, axis=0)
    excl1, excl2 = excl[:, :E], excl[:, E:]
    cnt1 = csum[:, :E]
    cnt2 = csum[:, E:]
    cnt_i = (cnt1 + cnt2).astype(jnp.int32)
    pc = (((cnt_i + (TM - 1)) // TM) * TM).astype(jnp.float32)
    ce = lax.broadcasted_iota(jnp.int32, (E, E), 0)
    de = lax.broadcasted_iota(jnp.int32, (E, E), 1)
    tri_e = (ce < de).astype(jnp.float32)
    offp = jnp.dot(pc, tri_e, preferred_element_type=jnp.float32)
    ends = offp + pc
    rank1 = jnp.sum(excl1 * oh1f, axis=1, keepdims=True)
    rank2 = jnp.sum((cnt1 + excl2) * oh2f, axis=1, keepdims=True)
    off1 = jnp.sum(offp * oh1f, axis=1, keepdims=True)
    off2 = jnp.sum(offp * oh2f, axis=1, keepdims=True)
    d1_ref[...] = (off1 + rank1).astype(jnp.int32)
    d2_ref[...] = (off2 + rank2).astype(jnp.int32)

    bpos = (lax.broadcasted_iota(jnp.int32, (1, NBE), 1) * TM).astype(
        jnp.float32)
    lane_e = lax.broadcasted_iota(jnp.int32, (1, E), 1)
    be = jnp.zeros((1, NBE), jnp.float32)
    for e in range(E):
        end_e = jnp.sum(jnp.where(lane_e == e, ends, 0.0), axis=1,
                        keepdims=True)
        be = be + (end_e <= bpos).astype(jnp.float32)
    be_i = jnp.minimum(be, float(E - 1)).astype(jnp.int32)
    tot = jnp.sum(pc, axis=1, keepdims=True).astype(jnp.int32) // TM
    lane_b = lax.broadcasted_iota(jnp.int32, (1, NBE), 1)
    be_ref[...] = jnp.where(lane_b == NBE - 1, tot, be_i)


def _router(x, w_gate):
    return pl.pallas_call(
        _router_body,
        out_shape=[
            jax.ShapeDtypeStruct((S, 1), jnp.int32),
            jax.ShapeDtypeStruct((S, 1), jnp.int32),
            jax.ShapeDtypeStruct((S, 1), jnp.float32),
            jax.ShapeDtypeStruct((S, 1), jnp.float32),
            jax.ShapeDtypeStruct((1, NBE), jnp.int32),
        ],
    )(x, w_gate)


TPW = S // NW
YC = 64


@functools.cache
def _sc_kernels():
    mesh = lambda: plsc.VectorSubcoreMesh(
        core_axis_name="c", subcore_axis_name="s")

    @functools.partial(
        pl.kernel,
        out_type=jax.ShapeDtypeStruct((NPR, D), jnp.float32),
        mesh=mesh(),
        scratch_types=[
            pltpu.VMEM((TPW,), jnp.int32),
            pltpu.VMEM((TPW,), jnp.int32),
            pltpu.VMEM((TPW, D), jnp.float32),
            pltpu.SemaphoreType.DMA,
        ],
    )
    def sc_dispatch(x_hbm, d1_hbm, d2_hbm, xs_hbm, d1_v, d2_v, buf_v, sem):
        wid = lax.axis_index("s") * 2 + lax.axis_index("c")
        tbase = wid * TPW
        pltpu.sync_copy(d1_hbm.at[pl.ds(tbase, TPW)], d1_v)
        pltpu.sync_copy(d2_hbm.at[pl.ds(tbase, TPW)], d2_v)
        pltpu.sync_copy(x_hbm.at[pl.ds(tbase, TPW)], buf_v)
        c1 = pltpu.async_copy(buf_v, xs_hbm.at[d1_v], sem)
        c2 = pltpu.async_copy(buf_v, xs_hbm.at[d2_v], sem)
        c1.wait()
        c2.wait()

    @functools.partial(
        pl.kernel,
        out_type=jax.ShapeDtypeStruct((PAIRS, D), jnp.float32),
        mesh=mesh(),
        scratch_types=[
            pltpu.VMEM((YC,), jnp.int32),
            pltpu.VMEM((YC, D), jnp.float32),
            pltpu.SemaphoreType.DMA,
        ],
    )
    def sc_gather_y(ys_hbm, d1_hbm, d2_hbm, y2_hbm, idx_v, buf_v, sem):
        wid = lax.axis_index("s") * 2 + lax.axis_index("c")
        tbase = wid * YC
        pltpu.sync_copy(d1_hbm.at[pl.ds(tbase, YC)], idx_v)
        pltpu.async_copy(ys_hbm.at[idx_v], buf_v, sem).wait()
        pltpu.sync_copy(buf_v, y2_hbm.at[pl.ds(tbase, YC)])
        pltpu.sync_copy(d2_hbm.at[pl.ds(tbase, YC)], idx_v)
        pltpu.async_copy(ys_hbm.at[idx_v], buf_v, sem).wait()
        pltpu.sync_copy(buf_v, y2_hbm.at[pl.ds(S + tbase, YC)])

    return sc_dispatch, sc_gather_y


NSP = 2
FH = FF // NSP


def _gmm_body(be_ref, x_ref, wga_ref, wgb_ref, wd_ref, y_ref):
    b = pl.program_id(0)
    j = pl.program_id(1)

    @pl.when(b < be_ref[NBE - 1])
    def _():
        xb = x_ref[...]
        a = jnp.dot(xb, wga_ref[0], preferred_element_type=jnp.float32)
        u = jnp.dot(xb, wgb_ref[0], preferred_element_type=jnp.float32)
        h = (a * (1.0 / (1.0 + jnp.exp(-a)))) * u
        part = jnp.dot(h, wd_ref[0], preferred_element_type=jnp.float32)

        @pl.when(j == 0)
        def _():
            y_ref[...] = part

        @pl.when(j != 0)
        def _():
            y_ref[...] = y_ref[...] + part


def _gmm(be, xs, w_gate_up, w_down):
    return pl.pallas_call(
        _gmm_body,
        grid_spec=pltpu.PrefetchScalarGridSpec(
            num_scalar_prefetch=1,
            grid=(NB, NSP),
            in_specs=[
                pl.BlockSpec(
                    (TM, D),
                    lambda b, j, be: (jnp.minimum(b, be[NBE - 1] - 1), 0)),
                pl.BlockSpec((1, D, FH), lambda b, j, be: (be[b], 0, j)),
                pl.BlockSpec((1, D, FH),
                             lambda b, j, be: (be[b], 0, j + NSP)),
                pl.BlockSpec((1, FH, D), lambda b, j, be: (be[b], j, 0)),
            ],
            out_specs=pl.BlockSpec(
                (TM, D),
                lambda b, j, be: (jnp.minimum(b, be[NBE - 1] - 1), 0)),
        ),
        out_shape=jax.ShapeDtypeStruct((NPR, D), jnp.float32),
    )(be, xs, w_gate_up, w_gate_up, w_down)


CB = 256


def _shared_body(x_ref, wgu_ref, wd_ref, o_ref):
    g = jnp.dot(x_ref[...], wgu_ref[...], preferred_element_type=jnp.float32,
                precision=lax.Precision.DEFAULT)
    a, b = g[:, :FF], g[:, FF:]
    h = (a * (1.0 / (1.0 + jnp.exp(-a)))) * b
    o_ref[...] = jnp.dot(h, wd_ref[...], preferred_element_type=jnp.float32,
                         precision=lax.Precision.DEFAULT)


def _shared_mlp(x, wgu_sh, wd_sh):
    nblk = S // CB
    return pl.pallas_call(
        _shared_body,
        grid=(nblk,),
        in_specs=[
            pl.BlockSpec((CB, D), lambda i: (i, 0)),
            pl.BlockSpec((D, 2 * FF), lambda i: (0, 0)),
            pl.BlockSpec((FF, D), lambda i: (0, 0)),
        ],
        out_specs=pl.BlockSpec((CB, D), lambda i: (i, 0)),
        out_shape=jax.ShapeDtypeStruct((S, D), jnp.float32),
    )(x, wgu_sh, wd_sh)


def _combine_body(sh_ref, ya_ref, yb_ref, w1_ref, w2_ref, o_ref):
    o_ref[...] = (sh_ref[...] + w1_ref[...] * ya_ref[...]
                  + w2_ref[...] * yb_ref[...])


def _combine(sh, y2, w1, w2):
    nblk = S // CB
    return pl.pallas_call(
        _combine_body,
        grid=(nblk,),
        in_specs=[
            pl.BlockSpec((CB, D), lambda i: (i, 0)),
            pl.BlockSpec((CB, D), lambda i: (i, 0)),
            pl.BlockSpec((CB, D), lambda i: (i + nblk, 0)),
            pl.BlockSpec((CB, 1), lambda i: (i, 0)),
            pl.BlockSpec((CB, 1), lambda i: (i, 0)),
        ],
        out_specs=pl.BlockSpec((CB, D), lambda i: (i, 0)),
        out_shape=jax.ShapeDtypeStruct((S, D), jnp.float32),
    )(sh, y2, y2, w1, w2)


def kernel(hidden_states, w_gate, w_gate_up, w_down, w_gate_up_shared,
           w_down_shared):
    x = hidden_states.reshape(S, D)
    d1, d2, w1, w2, be = _router(x, w_gate)
    d1r, d2r = d1.reshape(S), d2.reshape(S)
    sc_dispatch, sc_gather_y = _sc_kernels()
    xs = sc_dispatch(x, d1r, d2r)
    sh = _shared_mlp(x, w_gate_up_shared, w_down_shared)
    ys = _gmm(be.reshape(NBE), xs, w_gate_up, w_down)
    y2 = sc_gather_y(ys, d1r, d2r)
    out = _combine(sh, y2, w1, w2)
    return out.reshape(hidden_states.shape)


# speedup vs baseline: 1.2542x; 1.2542x over previous
import functools

import jax
import jax.numpy as jnp
from jax import lax
from jax.experimental import pallas as pl
from jax.experimental.pallas import tpu as pltpu
from jax.experimental.pallas import tpu_sc as plsc

S = 2048
D = 1024
E = 8
FF = 1024
TOPK = 2
PAIRS = S * TOPK
TM = 256
NPR = PAIRS + E * TM
NB = NPR // TM
NBE = 64
NW = 32
L = 16


def _router_body(x_ref, wg_ref, d1_ref, d2_ref, w1_ref, w2_ref, be_ref):
    x = x_ref[...]
    logits = jnp.dot(x, wg_ref[...], preferred_element_type=jnp.float32)
    lane = lax.broadcasted_iota(jnp.int32, (S, E), 1)
    l1 = jnp.max(logits, axis=1, keepdims=True)
    i1 = jnp.min(jnp.where(logits == l1, lane, E), axis=1, keepdims=True)
    oh1 = lane == i1
    masked = jnp.where(oh1, -jnp.inf, logits)
    l2 = jnp.max(masked, axis=1, keepdims=True)
    i2 = jnp.min(jnp.where(masked == l2, lane, E), axis=1, keepdims=True)
    oh2 = lane == i2
    w1 = 1.0 / (1.0 + jnp.exp(l2 - l1))
    w1_ref[...] = w1
    w2_ref[...] = 1.0 - w1

    oh1f = oh1.astype(jnp.float32)
    oh2f = oh2.astype(jnp.float32)
    ohb = jnp.concatenate([oh1f, oh2f], axis=1)
    CH = 128
    r = lax.broadcasted_iota(jnp.int32, (CH, CH), 0)
    c = lax.broadcasted_iota(jnp.int32, (CH, CH), 1)
    tri = (c < r).astype(jnp.float32)
    csum = jnp.zeros((1, 2 * E), jnp.float32)
    parts = []
    for k in range(S // CH):
        seg = ohb[k * CH:(k + 1) * CH, :]
        parts.append(jnp.dot(tri, seg, preferred_element_type=jnp.float32)
                     + csum)
        csum = csum + jnp.sum(seg, axis=0, keepdims=True)
    excl = jnp.concatenate(parts, axis=0)
    excl1, excl2 = excl[:, :E], excl[:, E:]
    cnt1 = csum[:, :E]
    cnt2 = csum[:, E:]
    cnt_i = (cnt1 + cnt2).astype(jnp.int32)
    pc = (((cnt_i + (TM - 1)) // TM) * TM).astype(jnp.float32)
    ce = lax.broadcasted_iota(jnp.int32, (E, E), 0)
    de = lax.broadcasted_iota(jnp.int32, (E, E), 1)
    tri_e = (ce < de).astype(jnp.float32)
    offp = jnp.dot(pc, tri_e, preferred_element_type=jnp.float32)
    ends = offp + pc
    rank1 = jnp.sum(excl1 * oh1f, axis=1, keepdims=True)
    rank2 = jnp.sum((cnt1 + excl2) * oh2f, axis=1, keepdims=True)
    off1 = jnp.sum(offp * oh1f, axis=1, keepdims=True)
    off2 = jnp.sum(offp * oh2f, axis=1, keepdims=True)
    d1_ref[...] = (off1 + rank1).astype(jnp.int32)
    d2_ref[...] = (off2 + rank2).astype(jnp.int32)

    bpos = (lax.broadcasted_iota(jnp.int32, (1, NBE), 1) * TM).astype(
        jnp.float32)
    lane_e = lax.broadcasted_iota(jnp.int32, (1, E), 1)
    be = jnp.zeros((1, NBE), jnp.float32)
    for e in range(E):
        end_e = jnp.sum(jnp.where(lane_e == e, ends, 0.0), axis=1,
                        keepdims=True)
        be = be + (end_e <= bpos).astype(jnp.float32)
    be_i = jnp.minimum(be, float(E - 1)).astype(jnp.int32)
    tot = jnp.sum(pc, axis=1, keepdims=True).astype(jnp.int32) // TM
    lane_b = lax.broadcasted_iota(jnp.int32, (1, NBE), 1)
    be_ref[...] = jnp.where(lane_b == NBE - 1, tot, be_i)


def _router(x, w_gate):
    return pl.pallas_call(
        _router_body,
        out_shape=[
            jax.ShapeDtypeStruct((S, 1), jnp.int32),
            jax.ShapeDtypeStruct((S, 1), jnp.int32),
            jax.ShapeDtypeStruct((S, 1), jnp.float32),
            jax.ShapeDtypeStruct((S, 1), jnp.float32),
            jax.ShapeDtypeStruct((1, NBE), jnp.int32),
        ],
    )(x, w_gate)


TPW = S // NW
YC = 64


@functools.cache
def _sc_kernels():
    mesh = lambda: plsc.VectorSubcoreMesh(
        core_axis_name="c", subcore_axis_name="s")

    @functools.partial(
        pl.kernel,
        out_type=jax.ShapeDtypeStruct((NPR, D), jnp.float32),
        mesh=mesh(),
        scratch_types=[
            pltpu.VMEM((TPW,), jnp.int32),
            pltpu.VMEM((TPW,), jnp.int32),
            pltpu.VMEM((TPW, D), jnp.float32),
            pltpu.SemaphoreType.DMA,
        ],
    )
    def sc_dispatch(x_hbm, d1_hbm, d2_hbm, xs_hbm, d1_v, d2_v, buf_v, sem):
        wid = lax.axis_index("s") * 2 + lax.axis_index("c")
        tbase = wid * TPW
        pltpu.sync_copy(d1_hbm.at[pl.ds(tbase, TPW)], d1_v)
        pltpu.sync_copy(d2_hbm.at[pl.ds(tbase, TPW)], d2_v)
        pltpu.sync_copy(x_hbm.at[pl.ds(tbase, TPW)], buf_v)
        c1 = pltpu.async_copy(buf_v, xs_hbm.at[d1_v], sem)
        c2 = pltpu.async_copy(buf_v, xs_hbm.at[d2_v], sem)
        c1.wait()
        c2.wait()

    @functools.partial(
        pl.kernel,
        out_type=jax.ShapeDtypeStruct((PAIRS, D), jnp.float32),
        mesh=mesh(),
        scratch_types=[
            pltpu.VMEM((YC,), jnp.int32),
            pltpu.VMEM((YC, D), jnp.float32),
            pltpu.SemaphoreType.DMA,
        ],
    )
    def sc_gather_y(ys_hbm, d1_hbm, d2_hbm, y2_hbm, idx_v, buf_v, sem):
        wid = lax.axis_index("s") * 2 + lax.axis_index("c")
        tbase = wid * YC
        pltpu.sync_copy(d1_hbm.at[pl.ds(tbase, YC)], idx_v)
        pltpu.async_copy(ys_hbm.at[idx_v], buf_v, sem).wait()
        pltpu.sync_copy(buf_v, y2_hbm.at[pl.ds(tbase, YC)])
        pltpu.sync_copy(d2_hbm.at[pl.ds(tbase, YC)], idx_v)
        pltpu.async_copy(ys_hbm.at[idx_v], buf_v, sem).wait()
        pltpu.sync_copy(buf_v, y2_hbm.at[pl.ds(S + tbase, YC)])

    return sc_dispatch, sc_gather_y


def _gmm_body(be_ref, x_ref, wgu_ref, wd_ref, y_ref):
    @pl.when(pl.program_id(0) < be_ref[NBE - 1])
    def _():
        xb = x_ref[...]
        g = jnp.dot(xb, wgu_ref[0], preferred_element_type=jnp.float32)
        a, b = g[:, :FF], g[:, FF:]
        h = (a * (1.0 / (1.0 + jnp.exp(-a)))) * b
        y_ref[...] = jnp.dot(h, wd_ref[0],
                             preferred_element_type=jnp.float32)


def _gmm(be, xs, w_gate_up, w_down):
    return pl.pallas_call(
        _gmm_body,
        grid_spec=pltpu.PrefetchScalarGridSpec(
            num_scalar_prefetch=1,
            grid=(NB,),
            in_specs=[
                pl.BlockSpec(
                    (TM, D),
                    lambda b, be: (jnp.minimum(b, be[NBE - 1] - 1), 0)),
                pl.BlockSpec((1, D, 2 * FF), lambda b, be: (be[b], 0, 0)),
                pl.BlockSpec((1, FF, D), lambda b, be: (be[b], 0, 0)),
            ],
            out_specs=pl.BlockSpec(
                (TM, D),
                lambda b, be: (jnp.minimum(b, be[NBE - 1] - 1), 0)),
        ),
        out_shape=jax.ShapeDtypeStruct((NPR, D), jnp.float32),
    )(be, xs, w_gate_up, w_down)


CB = 256


def _shared_body(x_ref, wgu_ref, wd_ref, o_ref):
    g = jnp.dot(x_ref[...], wgu_ref[...], preferred_element_type=jnp.float32,
                precision=lax.Precision.DEFAULT)
    a, b = g[:, :FF], g[:, FF:]
    h = (a * (1.0 / (1.0 + jnp.exp(-a)))) * b
    o_ref[...] = jnp.dot(h, wd_ref[...], preferred_element_type=jnp.float32,
                         precision=lax.Precision.DEFAULT)


def _shared_mlp(x, wgu_sh, wd_sh):
    nblk = S // CB
    return pl.pallas_call(
        _shared_body,
        grid=(nblk,),
        in_specs=[
            pl.BlockSpec((CB, D), lambda i: (i, 0)),
            pl.BlockSpec((D, 2 * FF), lambda i: (0, 0)),
            pl.BlockSpec((FF, D), lambda i: (0, 0)),
        ],
        out_specs=pl.BlockSpec((CB, D), lambda i: (i, 0)),
        out_shape=jax.ShapeDtypeStruct((S, D), jnp.float32),
    )(x, wgu_sh, wd_sh)


def _combine_body(sh_ref, ya_ref, yb_ref, w1_ref, w2_ref, o_ref):
    o_ref[...] = (sh_ref[...] + w1_ref[...] * ya_ref[...]
                  + w2_ref[...] * yb_ref[...])


def _combine(sh, y2, w1, w2):
    nblk = S // CB
    return pl.pallas_call(
        _combine_body,
        grid=(nblk,),
        in_specs=[
            pl.BlockSpec((CB, D), lambda i: (i, 0)),
            pl.BlockSpec((CB, D), lambda i: (i, 0)),
            pl.BlockSpec((CB, D), lambda i: (i + nblk, 0)),
            pl.BlockSpec((CB, 1), lambda i: (i, 0)),
            pl.BlockSpec((CB, 1), lambda i: (i, 0)),
        ],
        out_specs=pl.BlockSpec((CB, D), lambda i: (i, 0)),
        out_shape=jax.ShapeDtypeStruct((S, D), jnp.float32),
    )(sh, y2, y2, w1, w2)


def kernel(hidden_states, w_gate, w_gate_up, w_down, w_gate_up_shared,
           w_down_shared):
    x = hidden_states.reshape(S, D)
    d1, d2, w1, w2, be = _router(x, w_gate)
    d1r, d2r = d1.reshape(S), d2.reshape(S)
    sc_dispatch, sc_gather_y = _sc_kernels()
    xs = sc_dispatch(x, d1r, d2r)
    sh = _shared_mlp(x, w_gate_up_shared, w_down_shared)
    ys = _gmm(be.reshape(NBE), xs, w_gate_up, w_down)
    y2 = sc_gather_y(ys, d1r, d2r)
    out = _combine(sh, y2, w1, w2)
    return out.reshape(hidden_states.shape)


# speedup vs baseline: 1.2859x; 1.0253x over previous
import functools

import jax
import jax.numpy as jnp
from jax import lax
from jax.experimental import pallas as pl
from jax.experimental.pallas import tpu as pltpu
from jax.experimental.pallas import tpu_sc as plsc

S = 2048
D = 1024
E = 8
FF = 1024
TOPK = 2
PAIRS = S * TOPK
TM = 256
NPR = PAIRS + E * TM
NB = NPR // TM
NBE = 64
NW = 32
L = 16


def _router_body(x_ref, wg_ref, d1_ref, d2_ref, w1_ref, w2_ref, be_ref):
    x = x_ref[...]
    logits = jnp.dot(x, wg_ref[...], preferred_element_type=jnp.float32)
    lane = lax.broadcasted_iota(jnp.int32, (S, E), 1)
    l1 = jnp.max(logits, axis=1, keepdims=True)
    i1 = jnp.min(jnp.where(logits == l1, lane, E), axis=1, keepdims=True)
    oh1 = lane == i1
    masked = jnp.where(oh1, -jnp.inf, logits)
    l2 = jnp.max(masked, axis=1, keepdims=True)
    i2 = jnp.min(jnp.where(masked == l2, lane, E), axis=1, keepdims=True)
    oh2 = lane == i2
    w1 = 1.0 / (1.0 + jnp.exp(l2 - l1))
    w1_ref[...] = w1
    w2_ref[...] = 1.0 - w1

    oh1f = oh1.astype(jnp.float32)
    oh2f = oh2.astype(jnp.float32)
    ohb = jnp.concatenate([oh1f, oh2f], axis=1)
    CH = 128
    r = lax.broadcasted_iota(jnp.int32, (CH, CH), 0)
    c = lax.broadcasted_iota(jnp.int32, (CH, CH), 1)
    tri = (c < r).astype(jnp.float32)
    csum = jnp.zeros((1, 2 * E), jnp.float32)
    parts = []
    for k in range(S // CH):
        seg = ohb[k * CH:(k + 1) * CH, :]
        parts.append(jnp.dot(tri, seg, preferred_element_type=jnp.float32)
                     + csum)
        csum = csum + jnp.sum(seg, axis=0, keepdims=True)
    excl = jnp.concatenate(parts, axis=0)
    excl1, excl2 = excl[:, :E], excl[:, E:]
    cnt1 = csum[:, :E]
    cnt2 = csum[:, E:]
    cnt_i = (cnt1 + cnt2).astype(jnp.int32)
    pc = (((cnt_i + (TM - 1)) // TM) * TM).astype(jnp.float32)
    ce = lax.broadcasted_iota(jnp.int32, (E, E), 0)
    de = lax.broadcasted_iota(jnp.int32, (E, E), 1)
    tri_e = (ce < de).astype(jnp.float32)
    offp = jnp.dot(pc, tri_e, preferred_element_type=jnp.float32)
    ends = offp + pc
    rank1 = jnp.sum(excl1 * oh1f, axis=1, keepdims=True)
    rank2 = jnp.sum((cnt1 + excl2) * oh2f, axis=1, keepdims=True)
    off1 = jnp.sum(offp * oh1f, axis=1, keepdims=True)
    off2 = jnp.sum(offp * oh2f, axis=1, keepdims=True)
    d1_ref[...] = (off1 + rank1).astype(jnp.int32).reshape(S)
    d2_ref[...] = (off2 + rank2).astype(jnp.int32).reshape(S)

    bpos = (lax.broadcasted_iota(jnp.int32, (1, NBE), 1) * TM).astype(
        jnp.float32)
    lane_e = lax.broadcasted_iota(jnp.int32, (1, E), 1)
    be = jnp.zeros((1, NBE), jnp.float32)
    for e in range(E):
        end_e = jnp.sum(jnp.where(lane_e == e, ends, 0.0), axis=1,
                        keepdims=True)
        be = be + (end_e <= bpos).astype(jnp.float32)
    be_i = jnp.minimum(be, float(E - 1)).astype(jnp.int32)
    tot = jnp.sum(pc, axis=1, keepdims=True).astype(jnp.int32) // TM
    lane_b = lax.broadcasted_iota(jnp.int32, (1, NBE), 1)
    be_ref[...] = jnp.where(lane_b == NBE - 1, tot, be_i)


def _router(x, w_gate):
    return pl.pallas_call(
        _router_body,
        out_shape=[
            jax.ShapeDtypeStruct((S,), jnp.int32),
            jax.ShapeDtypeStruct((S,), jnp.int32),
            jax.ShapeDtypeStruct((S, 1), jnp.float32),
            jax.ShapeDtypeStruct((S, 1), jnp.float32),
            jax.ShapeDtypeStruct((1, NBE), jnp.int32),
        ],
    )(x, w_gate)


TPW = S // NW
YC = 64


@functools.cache
def _sc_kernels():
    mesh = lambda: plsc.VectorSubcoreMesh(
        core_axis_name="c", subcore_axis_name="s")

    @functools.partial(
        pl.kernel,
        out_type=jax.ShapeDtypeStruct((NPR, D), jnp.float32),
        mesh=mesh(),
        scratch_types=[
            pltpu.VMEM((TPW,), jnp.int32),
            pltpu.VMEM((TPW,), jnp.int32),
            pltpu.VMEM((TPW, D), jnp.float32),
            pltpu.SemaphoreType.DMA,
        ],
    )
    def sc_dispatch(x_hbm, d1_hbm, d2_hbm, xs_hbm, d1_v, d2_v, buf_v, sem):
        wid = lax.axis_index("s") * 2 + lax.axis_index("c")
        tbase = wid * TPW
        pltpu.sync_copy(d1_hbm.at[pl.ds(tbase, TPW)], d1_v)
        pltpu.sync_copy(d2_hbm.at[pl.ds(tbase, TPW)], d2_v)
        pltpu.sync_copy(x_hbm.at[pl.ds(tbase, TPW)], buf_v)
        c1 = pltpu.async_copy(buf_v, xs_hbm.at[d1_v], sem)
        c2 = pltpu.async_copy(buf_v, xs_hbm.at[d2_v], sem)
        c1.wait()
        c2.wait()

    @functools.partial(
        pl.kernel,
        out_type=jax.ShapeDtypeStruct((PAIRS, D), jnp.float32),
        mesh=mesh(),
        scratch_types=[
            pltpu.VMEM((YC,), jnp.int32),
            pltpu.VMEM((YC, D), jnp.float32),
            pltpu.SemaphoreType.DMA,
        ],
    )
    def sc_gather_y(ys_hbm, d1_hbm, d2_hbm, y2_hbm, idx_v, buf_v, sem):
        wid = lax.axis_index("s") * 2 + lax.axis_index("c")
        tbase = wid * YC
        pltpu.sync_copy(d1_hbm.at[pl.ds(tbase, YC)], idx_v)
        pltpu.async_copy(ys_hbm.at[idx_v], buf_v, sem).wait()
        pltpu.sync_copy(buf_v, y2_hbm.at[pl.ds(tbase, YC)])
        pltpu.sync_copy(d2_hbm.at[pl.ds(tbase, YC)], idx_v)
        pltpu.async_copy(ys_hbm.at[idx_v], buf_v, sem).wait()
        pltpu.sync_copy(buf_v, y2_hbm.at[pl.ds(S + tbase, YC)])

    return sc_dispatch, sc_gather_y


def _gmm_body(be_ref, x_ref, wgu_ref, wd_ref, y_ref):
    @pl.when(pl.program_id(0) < be_ref[NBE - 1])
    def _():
        xb = x_ref[...]
        g = jnp.dot(xb, wgu_ref[0], preferred_element_type=jnp.float32)
        a, b = g[:, :FF], g[:, FF:]
        h = (a * (1.0 / (1.0 + jnp.exp(-a)))) * b
        y_ref[...] = jnp.dot(h, wd_ref[0],
                             preferred_element_type=jnp.float32)


def _gmm(be, xs, w_gate_up, w_down):
    return pl.pallas_call(
        _gmm_body,
        grid_spec=pltpu.PrefetchScalarGridSpec(
            num_scalar_prefetch=1,
            grid=(NB,),
            in_specs=[
                pl.BlockSpec(
                    (TM, D),
                    lambda b, be: (jnp.minimum(b, be[NBE - 1] - 1), 0)),
                pl.BlockSpec((1, D, 2 * FF), lambda b, be: (be[b], 0, 0)),
                pl.BlockSpec((1, FF, D), lambda b, be: (be[b], 0, 0)),
            ],
            out_specs=pl.BlockSpec(
                (TM, D),
                lambda b, be: (jnp.minimum(b, be[NBE - 1] - 1), 0)),
        ),
        out_shape=jax.ShapeDtypeStruct((NPR, D), jnp.float32),
    )(be, xs, w_gate_up, w_down)


CB = 256


def _shared_body(x_ref, wgu_ref, wd_ref, o_ref):
    g = jnp.dot(x_ref[...], wgu_ref[...], preferred_element_type=jnp.float32,
                precision=lax.Precision.DEFAULT)
    a, b = g[:, :FF], g[:, FF:]
    h = (a * (1.0 / (1.0 + jnp.exp(-a)))) * b
    o_ref[...] = jnp.dot(h, wd_ref[...], preferred_element_type=jnp.float32,
                         precision=lax.Precision.DEFAULT)


def _shared_mlp(x, wgu_sh, wd_sh):
    nblk = S // CB
    return pl.pallas_call(
        _shared_body,
        grid=(nblk,),
        in_specs=[
            pl.BlockSpec((CB, D), lambda i: (i, 0)),
            pl.BlockSpec((D, 2 * FF), lambda i: (0, 0)),
            pl.BlockSpec((FF, D), lambda i: (0, 0)),
        ],
        out_specs=pl.BlockSpec((CB, D), lambda i: (i, 0)),
        out_shape=jax.ShapeDtypeStruct((S, D), jnp.float32),
    )(x, wgu_sh, wd_sh)


def _combine_body(sh_ref, ya_ref, yb_ref, w1_ref, w2_ref, o_ref):
    o_ref[...] = (sh_ref[...] + w1_ref[...] * ya_ref[...]
                  + w2_ref[...] * yb_ref[...])


def _combine(sh, y2, w1, w2):
    nblk = S // CB
    return pl.pallas_call(
        _combine_body,
        grid=(nblk,),
        in_specs=[
            pl.BlockSpec((CB, D), lambda i: (i, 0)),
            pl.BlockSpec((CB, D), lambda i: (i, 0)),
            pl.BlockSpec((CB, D), lambda i: (i + nblk, 0)),
            pl.BlockSpec((CB, 1), lambda i: (i, 0)),
            pl.BlockSpec((CB, 1), lambda i: (i, 0)),
        ],
        out_specs=pl.BlockSpec((CB, D), lambda i: (i, 0)),
        out_shape=jax.ShapeDtypeStruct((S, D), jnp.float32),
    )(sh, y2, y2, w1, w2)


def kernel(hidden_states, w_gate, w_gate_up, w_down, w_gate_up_shared,
           w_down_shared):
    x = hidden_states.reshape(S, D)
    d1r, d2r, w1, w2, be = _router(x, w_gate)
    sc_dispatch, sc_gather_y = _sc_kernels()
    xs = sc_dispatch(x, d1r, d2r)
    sh = _shared_mlp(x, w_gate_up_shared, w_down_shared)
    ys = _gmm(be.reshape(NBE), xs, w_gate_up, w_down)
    y2 = sc_gather_y(ys, d1r, d2r)
    out = _combine(sh, y2, w1, w2)
    return out.reshape(hidden_states.shape)
